# baseline (speedup 1.0000x reference)
import functools

import jax
import jax.numpy as jnp
import numpy as np
from jax import lax
from jax.experimental import pallas as pl
from jax.experimental.pallas import tpu as pltpu
from jax.experimental.pallas import tpu_sc as plsc

_F = 26
_D = 64
_B = 4096
_P = _F * (_F - 1) // 2
_EMBED_OUT = _F * _D
_HX = 384

_rows, _cols = np.triu_indices(_F, k=1)
_pid = np.zeros((_F, _F), dtype=np.int32)
for _p, (_r, _c) in enumerate(zip(_rows, _cols)):
    _pid[_r, _c] = _p
_src = np.zeros((_P,), dtype=np.int32)
_fi = np.zeros((_P,), dtype=np.int32)
_gi = np.zeros((_P,), dtype=np.int32)
_j = 0
for _k in range(1, _F):
    for _f in range(_F - _k):
        _src[_j] = _pid[_f, _f + _k]
        _fi[_j] = _f
        _gi[_j] = _f + _k
        _j += 1
_oh_f = np.zeros((_P, _F), dtype=np.float32)
_oh_g = np.zeros((_P, _F), dtype=np.float32)
_oh_f[np.arange(_P), _fi] = 1.0
_oh_g[np.arange(_P), _gi] = 1.0
_SEGN = 32
_sel = (np.arange(_EMBED_OUT)[:, None] // _D
        == np.arange(_SEGN)[None, :]).astype(np.float32)
_NC = 2
_NS = 16
_NW = _NC * _NS
_ROWS = _B * _F
_B_PER_W = _ROWS // _NW
_N_CHUNK = 4
_CHUNK = _B_PER_W // _N_CHUNK


def _sc_gather_body(table_hbm, idx_hbm, out_hbm, idx0, idx1, rows0, rows1,
                    sem0, sem1):
    wid = lax.axis_index("s") * _NC + lax.axis_index("c")
    base = wid * _B_PER_W
    bufs = ((idx0, rows0, sem0), (idx1, rows1, sem1))
    pending = [None, None]
    offs = [None, None]
    for c in range(_N_CHUNK):
        s = c % 2
        iv, rv, sm = bufs[s]
        if pending[s] is not None:
            pending[s].wait()
            pltpu.sync_copy(rv, out_hbm.at[pl.ds(offs[s], _CHUNK)])
        off = base + c * _CHUNK
        pltpu.sync_copy(idx_hbm.at[pl.ds(off, _CHUNK)], iv)
        pending[s] = pltpu.async_copy(table_hbm.at[iv], rv, sm)
        offs[s] = off
    for s in range(2):
        iv, rv, sm = bufs[s]
        pending[s].wait()
        pltpu.sync_copy(rv, out_hbm.at[pl.ds(offs[s], _CHUNK)])


@functools.cache
def _sc_gather_kernel():
    return pl.kernel(
        _sc_gather_body,
        mesh=plsc.VectorSubcoreMesh(core_axis_name="c", subcore_axis_name="s"),
        out_type=jax.ShapeDtypeStruct((_ROWS, _D), jnp.float32),
        scratch_types=[
            pltpu.VMEM((_CHUNK,), jnp.int32),
            pltpu.VMEM((_CHUNK,), jnp.int32),
            pltpu.VMEM((_CHUNK, _D), jnp.float32),
            pltpu.VMEM((_CHUNK, _D), jnp.float32),
            pltpu.SemaphoreType.DMA,
            pltpu.SemaphoreType.DMA,
        ],
        compiler_params=pltpu.CompilerParams(use_tc_tiling_on_sc=False),
    )


def _sc_gather(table, idx):
    return _sc_gather_kernel()(table, idx)


_BT = 256


def _tc_body(xvf_ref, sel_ref, w1f_ref, w1p_ref, b1_ref, w2_ref, b2_ref,
             w3_ref, b3_ref, wo_ref, bo_ref, out_ref):
    flat32 = xvf_ref[...]
    flat = flat32.astype(jnp.bfloat16)
    acc = jnp.dot(flat, w1f_ref[...], preferred_element_type=jnp.float32)
    pieces = []
    for k in range(1, _F):
        w = _EMBED_OUT - _D * k
        m = (flat32[:, :w] * flat32[:, _D * k:]).astype(jnp.bfloat16)
        r = jnp.dot(m, sel_ref[:w, :], preferred_element_type=jnp.float32)
        pieces.append(r[:, :_F - k])
    pieces.append(jnp.zeros((_BT, _HX - _P), jnp.float32))
    hx = jnp.concatenate(pieces, axis=1).astype(jnp.bfloat16)
    acc = acc + jnp.dot(hx, w1p_ref[...], preferred_element_type=jnp.float32)
    h = jnp.maximum(acc + b1_ref[...], 0.0).astype(jnp.bfloat16)
    h = jnp.maximum(
        jnp.dot(h, w2_ref[...], preferred_element_type=jnp.float32)
        + b2_ref[...], 0.0).astype(jnp.bfloat16)
    h = jnp.maximum(
        jnp.dot(h, w3_ref[...], preferred_element_type=jnp.float32)
        + b3_ref[...], 0.0).astype(jnp.bfloat16)
    out_ref[...] = (
        jnp.dot(h, wo_ref[...], preferred_element_type=jnp.float32)
        + bo_ref[...])


def _tc_mlp(xvf, sel, w1f, w1p, b1, w2, b2, w3, b3, wo, bo):
    grid = (_B // _BT,)
    full = lambda shape: pl.BlockSpec(shape, lambda i: (0,) * len(shape))
    return pl.pallas_call(
        _tc_body,
        grid=grid,
        in_specs=[
            pl.BlockSpec((_BT, _EMBED_OUT), lambda i: (i, 0)),
            full(sel.shape),
            full(w1f.shape),
            full(w1p.shape),
            full(b1.shape),
            full(w2.shape),
            full(b2.shape),
            full(w3.shape),
            full(b3.shape),
            full(wo.shape),
            full(bo.shape),
        ],
        out_specs=pl.BlockSpec((_BT, 1), lambda i: (i, 0)),
        out_shape=jax.ShapeDtypeStruct((_B, 1), jnp.float32),
        compiler_params=pltpu.CompilerParams(
            dimension_semantics=("arbitrary",)),
    )(xvf, sel, w1f, w1p, b1, w2, b2, w3, b3, wo, bo)


def kernel(x, beta, arch, embedding, W1, b1, W2, b2, W3, b3, Wo, bo):
    prob = jax.nn.softmax(arch / beta, axis=0)
    w1f = W1[:_EMBED_OUT] * jnp.repeat(prob, _D)[:, None]
    scale = (_oh_f @ prob) * (_oh_g @ prob)
    w1p = jnp.concatenate(
        [W1[_EMBED_OUT:][_src] * scale[:, None],
         jnp.zeros((_HX - _P, W1.shape[1]), jnp.float32)], axis=0)

    idx = x.reshape(-1).astype(jnp.int32)
    gathered = _sc_gather(embedding, idx)
    xvf = gathered.reshape(_B, _EMBED_OUT)

    out = _tc_mlp(
        xvf, jnp.asarray(_sel, jnp.bfloat16),
        w1f.astype(jnp.bfloat16), w1p.astype(jnp.bfloat16),
        b1.reshape(1, -1), W2.astype(jnp.bfloat16), b2.reshape(1, -1),
        W3.astype(jnp.bfloat16), b3.reshape(1, -1),
        Wo.astype(jnp.bfloat16), bo.reshape(1, 1))
    return out[:, 0]


# speedup vs baseline: 1.0027x; 1.0027x over previous
import functools

import jax
import jax.numpy as jnp
import numpy as np
from jax import lax
from jax.experimental import pallas as pl
from jax.experimental.pallas import tpu as pltpu
from jax.experimental.pallas import tpu_sc as plsc

_F = 26
_D = 64
_B = 4096
_P = _F * (_F - 1) // 2
_EMBED_OUT = _F * _D
_HX = 384

_rows, _cols = np.triu_indices(_F, k=1)
_pid = np.zeros((_F, _F), dtype=np.int32)
for _p, (_r, _c) in enumerate(zip(_rows, _cols)):
    _pid[_r, _c] = _p
_src = np.zeros((_P,), dtype=np.int32)
_fi = np.zeros((_P,), dtype=np.int32)
_gi = np.zeros((_P,), dtype=np.int32)
_j = 0
for _k in range(1, _F):
    for _f in range(_F - _k):
        _src[_j] = _pid[_f, _f + _k]
        _fi[_j] = _f
        _gi[_j] = _f + _k
        _j += 1
_oh_f = np.zeros((_P, _F), dtype=np.float32)
_oh_g = np.zeros((_P, _F), dtype=np.float32)
_oh_f[np.arange(_P), _fi] = 1.0
_oh_g[np.arange(_P), _gi] = 1.0
_SEGN = 32
_sel = (np.arange(_EMBED_OUT)[:, None] // _D
        == np.arange(_SEGN)[None, :]).astype(np.float32)
_NC = 2
_NS = 16
_NW = _NC * _NS
_ROWS = _B * _F
_B_PER_W = _ROWS // _NW
_N_CHUNK = 2
_CHUNK = _B_PER_W // _N_CHUNK


def _sc_gather_body(table_hbm, idx_hbm, out_hbm, idx_v, rows_v, sem):
    wid = lax.axis_index("s") * _NC + lax.axis_index("c")
    base = wid * _B_PER_W
    for c in range(_N_CHUNK):
        off = base + c * _CHUNK
        pltpu.sync_copy(idx_hbm.at[pl.ds(off, _CHUNK)], idx_v)
        pltpu.async_copy(table_hbm.at[idx_v], rows_v, sem).wait()
        pltpu.sync_copy(rows_v, out_hbm.at[pl.ds(off, _CHUNK)])


@functools.cache
def _sc_gather_kernel():
    return pl.kernel(
        _sc_gather_body,
        mesh=plsc.VectorSubcoreMesh(core_axis_name="c", subcore_axis_name="s"),
        out_type=jax.ShapeDtypeStruct((_ROWS, _D), jnp.float32),
        scratch_types=[
            pltpu.VMEM((_CHUNK,), jnp.int32),
            pltpu.VMEM((_CHUNK, _D), jnp.float32),
            pltpu.SemaphoreType.DMA,
        ],
        compiler_params=pltpu.CompilerParams(use_tc_tiling_on_sc=False),
    )


def _sc_gather(table, idx):
    return _sc_gather_kernel()(table, idx)


_BT = 256


def _tc_body(xvf_ref, sel_ref, w1f_ref, w1p_ref, b1_ref, w2_ref, b2_ref,
             w3_ref, b3_ref, wo_ref, bo_ref, out_ref):
    flat32 = xvf_ref[...]
    flat = flat32.astype(jnp.bfloat16)
    acc = jnp.dot(flat, w1f_ref[...], preferred_element_type=jnp.float32)
    pieces = []
    for k in range(1, _F):
        w = _EMBED_OUT - _D * k
        m = (flat32[:, :w] * flat32[:, _D * k:]).astype(jnp.bfloat16)
        r = jnp.dot(m, sel_ref[:w, :], preferred_element_type=jnp.float32)
        pieces.append(r[:, :_F - k])
    pieces.append(jnp.zeros((_BT, _HX - _P), jnp.float32))
    hx = jnp.concatenate(pieces, axis=1).astype(jnp.bfloat16)
    acc = acc + jnp.dot(hx, w1p_ref[...], preferred_element_type=jnp.float32)
    h = jnp.maximum(acc + b1_ref[...], 0.0).astype(jnp.bfloat16)
    h = jnp.maximum(
        jnp.dot(h, w2_ref[...], preferred_element_type=jnp.float32)
        + b2_ref[...], 0.0).astype(jnp.bfloat16)
    h = jnp.maximum(
        jnp.dot(h, w3_ref[...], preferred_element_type=jnp.float32)
        + b3_ref[...], 0.0).astype(jnp.bfloat16)
    out_ref[...] = (
        jnp.dot(h, wo_ref[...], preferred_element_type=jnp.float32)
        + bo_ref[...])


def _tc_mlp(xvf, sel, w1f, w1p, b1, w2, b2, w3, b3, wo, bo):
    grid = (_B // _BT,)
    full = lambda shape: pl.BlockSpec(shape, lambda i: (0,) * len(shape))
    return pl.pallas_call(
        _tc_body,
        grid=grid,
        in_specs=[
            pl.BlockSpec((_BT, _EMBED_OUT), lambda i: (i, 0)),
            full(sel.shape),
            full(w1f.shape),
            full(w1p.shape),
            full(b1.shape),
            full(w2.shape),
            full(b2.shape),
            full(w3.shape),
            full(b3.shape),
            full(wo.shape),
            full(bo.shape),
        ],
        out_specs=pl.BlockSpec((_BT, 1), lambda i: (i, 0)),
        out_shape=jax.ShapeDtypeStruct((_B, 1), jnp.float32),
        compiler_params=pltpu.CompilerParams(
            dimension_semantics=("arbitrary",)),
    )(xvf, sel, w1f, w1p, b1, w2, b2, w3, b3, wo, bo)


def kernel(x, beta, arch, embedding, W1, b1, W2, b2, W3, b3, Wo, bo):
    prob = jax.nn.softmax(arch / beta, axis=0)
    w1f = W1[:_EMBED_OUT] * jnp.repeat(prob, _D)[:, None]
    scale = (_oh_f @ prob) * (_oh_g @ prob)
    w1p = jnp.concatenate(
        [W1[_EMBED_OUT:][_src] * scale[:, None],
         jnp.zeros((_HX - _P, W1.shape[1]), jnp.float32)], axis=0)

    idx = x.reshape(-1).astype(jnp.int32)
    gathered = _sc_gather(embedding, idx)
    xvf = gathered.reshape(_B, _EMBED_OUT)

    out = _tc_mlp(
        xvf, jnp.asarray(_sel, jnp.bfloat16),
        w1f.astype(jnp.bfloat16), w1p.astype(jnp.bfloat16),
        b1.reshape(1, -1), W2.astype(jnp.bfloat16), b2.reshape(1, -1),
        W3.astype(jnp.bfloat16), b3.reshape(1, -1),
        Wo.astype(jnp.bfloat16), bo.reshape(1, 1))
    return out[:, 0]
